# baseline (speedup 1.0000x reference)
import functools

import jax
import jax.numpy as jnp
from jax.experimental import pallas as pl
from jax.experimental.pallas import tpu as pltpu

N = 10000
E = 160000
G = 512
H = 512
L = 128
DEC = 512
EPS = 1e-5


def _ahat_body(zi_ref, zj_ref, out_ref):
    ip = jax.lax.dot_general(
        zi_ref[...], zj_ref[...], (((1,), (1,)), ((), ())),
        preferred_element_type=jnp.float32)
    ip = jnp.clip(ip, -10.0, 10.0)
    a = jax.nn.sigmoid(ip)
    out_ref[...] = jnp.clip(a, 1e-7, 1.0 - 1e-7)


def _ahat(zn):
    n = zn.shape[0]
    bm = 1024
    bn = 1024
    grid = (pl.cdiv(n, bm), pl.cdiv(n, bn))
    return pl.pallas_call(
        _ahat_body,
        grid=grid,
        in_specs=[
            pl.BlockSpec((bm, L), lambda i, j: (i, 0)),
            pl.BlockSpec((bn, L), lambda i, j: (j, 0)),
        ],
        out_specs=pl.BlockSpec((bm, bn), lambda i, j: (i, j)),
        out_shape=jax.ShapeDtypeStruct((n, n), jnp.float32),
    )(zn, zn)


def _spmm(ei, w, h, n):
    src = ei[0]
    dst = ei[1]
    msg = h[src] * w[:, None]
    return jax.ops.segment_sum(msg, dst, num_segments=n)


def _gcn_encoder(x, ei, w, p, name):
    h = x @ p[name + '_W1'] + p[name + '_b1']
    h = _spmm(ei, w, h, x.shape[0])
    h = jax.nn.relu(h)
    z = h @ p[name + '_W2'] + p[name + '_b2']
    z = _spmm(ei, w, z, x.shape[0])
    return z


def kernel(x, ei_knn, ei_mnn, ei_cluster, w_knn, w_mnn, w_cluster, params):
    p = params
    Z_knn = _gcn_encoder(x, ei_knn, w_knn, p, 'knn')
    Z_mnn = _gcn_encoder(x, ei_mnn, w_mnn, p, 'mnn')
    Z_cluster = _gcn_encoder(x, ei_cluster, w_cluster, p, 'cluster')
    z_fused = jax.nn.relu(
        jnp.concatenate([Z_knn, Z_mnn, Z_cluster], axis=1) @ p['fuse_W']
        + p['fuse_b'])
    h = jax.nn.relu(z_fused @ p['dec_W1'] + p['dec_b1'])
    mean = jnp.mean(h, axis=0)
    var = jnp.var(h, axis=0)
    h = (h - mean) / jnp.sqrt(var + EPS) * p['bn_gamma'] + p['bn_beta']
    h = jax.nn.relu(h @ p['dec_W2'] + p['dec_b2'])
    mu = jnp.exp(jnp.clip(h @ p['mu_W'] + p['mu_b'], -15.0, 15.0))
    theta = jnp.clip(jax.nn.softplus(h @ p['th_W'] + p['th_b']), 1e-4, 1e4)
    pi = jax.nn.sigmoid(h @ p['pi_W'] + p['pi_b'])
    zc = jnp.concatenate([Z_knn, Z_mnn], axis=1)
    hc = jax.nn.relu(zc @ p['cv_W1'] + p['cv_b1'])
    Z_final = hc @ p['cv_W2'] + p['cv_b2']
    Zn = Z_final / jnp.clip(
        jnp.linalg.norm(Z_final, axis=1, keepdims=True), 1e-12, None)
    A_hat = _ahat(Zn)
    return mu, theta, pi, A_hat, Z_final, Z_knn, Z_mnn, Z_cluster


# speedup vs baseline: 1.8555x; 1.8555x over previous
import functools

import jax
import jax.numpy as jnp
from jax import lax
from jax.experimental import pallas as pl
from jax.experimental.pallas import tpu as pltpu
from jax.experimental.pallas import tpu_sc as plsc

N = 10000
E = 160000
G = 512
H = 512
L = 128
DEC = 512
EPS = 1e-5

NCORES = 2
NSUB = 16
ROWS_PER_SUB = 632
NP = NSUB * ROWS_PER_SUB


def _spmm_body(n_items, vdiv, tdv, nchunks, wmode,
               tables, srcp, dstp, wp, zeros, out,
               acc, src_buf, dst_buf, w_buf, rows, gsem):
    cid = lax.axis_index("c")
    sid = lax.axis_index("s")
    widx = sid if wmode == 16 else cid * 16 + sid

    def item_step(t, carry):
        i = 2 * t + cid
        v = i // vdiv
        tbase = (i // tdv) * N
        pltpu.sync_copy(zeros, acc.at[pl.ds(sid * ROWS_PER_SUB, ROWS_PER_SUB)])
        pltpu.sync_copy(srcp.at[v, widx], src_buf)
        pltpu.sync_copy(dstp.at[v, widx], dst_buf)
        pltpu.sync_copy(wp.at[v, widx], w_buf)

        def off_r(r, c):
            for tt in range(8):
                sl = pl.ds(tt * 16, 16)
                src_buf[r, 0, sl] = src_buf[r, 0, sl] + tbase
            return c
        lax.fori_loop(0, nchunks, off_r, 0)
        plsc.subcore_barrier()

        def chunk(j, c):
            pltpu.async_copy(tables.at[src_buf.at[j, 0]], rows, gsem).wait()

            def scale(kk, c2):
                wv = w_buf[j, 0, pl.ds(kk * 16, 16)]
                for l in range(16):
                    wk = wv[l]
                    row = kk * 16 + l
                    for tt in range(8):
                        sl = pl.ds(tt * 16, 16)
                        rows[row, sl] = rows[row, sl] * wk
                return c2
            lax.fori_loop(0, 8, scale, 0)
            pltpu.sync_copy(rows, acc.at[dst_buf.at[j, 0]], add=True)
            return c
        lax.fori_loop(0, nchunks, chunk, 0)
        plsc.subcore_barrier()
        pltpu.sync_copy(
            acc.at[pl.ds(sid * ROWS_PER_SUB, ROWS_PER_SUB)],
            out.at[pl.ds(i * NP + sid * ROWS_PER_SUB, ROWS_PER_SUB)])
        return carry

    lax.fori_loop(0, n_items // 2, item_step, 0)


def _make_spmm(n_items, vdiv, tdv, nchunks, wmode, n_tables):
    body = functools.partial(_spmm_body, n_items, vdiv, tdv, nchunks, wmode)
    return pl.kernel(
        body,
        out_type=jax.ShapeDtypeStruct((n_items * NP, 128), jnp.float32),
        mesh=plsc.VectorSubcoreMesh(core_axis_name="c", subcore_axis_name="s"),
        scratch_types=[
            pltpu.VMEM_SHARED((NP, 128), jnp.float32),
            pltpu.VMEM((nchunks, 1, 128), jnp.int32),
            pltpu.VMEM((nchunks, 1, 128), jnp.int32),
            pltpu.VMEM((nchunks, 1, 128), jnp.float32),
            pltpu.VMEM((128, 128), jnp.float32),
            pltpu.SemaphoreType.DMA,
        ],
    )


def _pad_edges(ei_list, w_list, nworkers, nchunks):
    per = E // nworkers
    padded = nchunks * 128
    pad = padded - per
    srcs, dsts, ws = [], [], []
    spread = (jnp.arange(pad, dtype=jnp.int32) * 389) % N
    for ei, w in zip(ei_list, w_list):
        s = ei[0].reshape(nworkers, per)
        d = ei[1].reshape(nworkers, per)
        wv = w.reshape(nworkers, per)
        s = jnp.pad(s, ((0, 0), (0, pad)))
        d = jnp.concatenate(
            [d, jnp.broadcast_to(spread, (nworkers, pad))], axis=1)
        wv = jnp.pad(wv, ((0, 0), (0, pad)))
        srcs.append(s.reshape(nworkers, nchunks, 1, 128))
        dsts.append(d.reshape(nworkers, nchunks, 1, 128))
        ws.append(wv.reshape(nworkers, nchunks, 1, 128))
    return (jnp.stack(srcs).astype(jnp.int32),
            jnp.stack(dsts).astype(jnp.int32),
            jnp.stack(ws).astype(jnp.float32))


def _ahat_body(zi_ref, zj_ref, out_ref):
    ip = jax.lax.dot_general(
        zi_ref[...], zj_ref[...], (((1,), (1,)), ((), ())),
        preferred_element_type=jnp.float32)
    ip = jnp.clip(ip, -10.0, 10.0)
    a = jax.nn.sigmoid(ip)
    out_ref[...] = jnp.clip(a, 1e-7, 1.0 - 1e-7)


def _ahat(zn):
    n = zn.shape[0]
    bm = 1024
    bn = 1024
    grid = (pl.cdiv(n, bm), pl.cdiv(n, bn))
    return pl.pallas_call(
        _ahat_body,
        grid=grid,
        in_specs=[
            pl.BlockSpec((bm, L), lambda i, j: (i, 0)),
            pl.BlockSpec((bn, L), lambda i, j: (j, 0)),
        ],
        out_specs=pl.BlockSpec((bm, bn), lambda i, j: (i, j)),
        out_shape=jax.ShapeDtypeStruct((n, n), jnp.float32),
    )(zn, zn)


def kernel(x, ei_knn, ei_mnn, ei_cluster, w_knn, w_mnn, w_cluster, params):
    p = params
    eis = [ei_knn, ei_mnn, ei_cluster]
    ws = [w_knn, w_mnn, w_cluster]
    names = ['knn', 'mnn', 'cluster']
    zeros = jnp.zeros((ROWS_PER_SUB, 128), jnp.float32)

    h_all = jnp.stack(
        [x @ p[n_ + '_W1'] + p[n_ + '_b1'] for n_ in names])
    tables1 = h_all.reshape(3, N, 4, 128).transpose(0, 2, 1, 3)
    tables1 = tables1.reshape(12 * N, 128)

    src1, dst1, w1 = _pad_edges(eis, ws, NSUB, 79)
    spmm1 = _make_spmm(12, 4, 1, 79, 16, 12)
    s1 = spmm1(tables1, src1, dst1, w1, zeros)
    s1 = s1.reshape(3, 4, NP, 128)[:, :, :N]
    s1 = s1.transpose(0, 2, 1, 3).reshape(3, N, 512)

    hr = jax.nn.relu(s1)
    z_all = jnp.stack(
        [hr[i] @ p[n_ + '_W2'] + p[n_ + '_b2'] for i, n_ in enumerate(names)])
    tables2 = z_all.reshape(3 * N, 128)

    src2, dst2, w2 = _pad_edges(eis, ws, 2 * NSUB, 40)
    spmm2 = _make_spmm(6, 2, 2, 40, 32, 3)
    s2 = spmm2(tables2, src2, dst2, w2, zeros)
    s2 = s2.reshape(3, 2, NP, 128)[:, :, :N]
    Z_knn = s2[0, 0] + s2[0, 1]
    Z_mnn = s2[1, 0] + s2[1, 1]
    Z_cluster = s2[2, 0] + s2[2, 1]

    z_fused = jax.nn.relu(
        jnp.concatenate([Z_knn, Z_mnn, Z_cluster], axis=1) @ p['fuse_W']
        + p['fuse_b'])
    h = jax.nn.relu(z_fused @ p['dec_W1'] + p['dec_b1'])
    mean = jnp.mean(h, axis=0)
    var = jnp.var(h, axis=0)
    h = (h - mean) / jnp.sqrt(var + EPS) * p['bn_gamma'] + p['bn_beta']
    h = jax.nn.relu(h @ p['dec_W2'] + p['dec_b2'])
    mu = jnp.exp(jnp.clip(h @ p['mu_W'] + p['mu_b'], -15.0, 15.0))
    theta = jnp.clip(jax.nn.softplus(h @ p['th_W'] + p['th_b']), 1e-4, 1e4)
    pi = jax.nn.sigmoid(h @ p['pi_W'] + p['pi_b'])

    zc = jnp.concatenate([Z_knn, Z_mnn], axis=1)
    hc = jax.nn.relu(zc @ p['cv_W1'] + p['cv_b1'])
    Z_final = hc @ p['cv_W2'] + p['cv_b2']
    Zn = Z_final / jnp.clip(
        jnp.linalg.norm(Z_final, axis=1, keepdims=True), 1e-12, None)
    A_hat = _ahat(Zn)
    return mu, theta, pi, A_hat, Z_final, Z_knn, Z_mnn, Z_cluster
